# Initial kernel scaffold; baseline (speedup 1.0000x reference)
import jax
import jax.numpy as jnp
from jax.experimental import pallas as pl


def kernel(x_nchw, w1, b1, w2, b2):
    raise NotImplementedError("write your pallas kernel here")


# speedup vs baseline: 1.5546x; 1.5546x over previous
import jax
import jax.numpy as jnp
from jax.experimental import pallas as pl
from jax.experimental.pallas import tpu as pltpu

_VMEM_LIMIT_BYTES = 64 * 1024 * 1024


def _make_body(hw_true):
    inv_hw = 1.0 / float(hw_true)

    def body(x_ref, w1a_ref, w1b_ref, b1_ref, w2_ref, b2_ref, o_ref):
        x = x_ref[...]
        avg = jnp.sum(x, axis=-1) * inv_hw
        mx = jnp.max(x, axis=-1)

        h = (jnp.dot(avg, w1a_ref[...], preferred_element_type=jnp.float32)
             + jnp.dot(mx, w1b_ref[...], preferred_element_type=jnp.float32)
             + b1_ref[...])
        h = jnp.maximum(h, 0.0)
        logits = (jnp.dot(h, w2_ref[...], preferred_element_type=jnp.float32)
                  + b2_ref[...])
        scale = jax.nn.sigmoid(logits)
        o_ref[...] = x * scale[:, :, None]

    return body


def kernel(x_nchw, w1, b1, w2, b2):
    N, C, H, W = x_nchw.shape
    HW = H * W
    hid = w1.shape[1]

    w1a = w1[:C, :]
    w1b = w1[C:, :]
    b1_2d = b1.reshape(1, hid)
    b2_2d = b2.reshape(1, C)

    x_flat = x_nchw.reshape(N, C, HW).astype(jnp.float32)

    for nb in (8, 4, 2, 1):
        if N % nb == 0:
            NB = nb
            break

    cost = pl.CostEstimate(
        flops=int(N * (2 * 2 * C * hid + 2 * hid * C) + 2 * N * C * HW),
        transcendentals=int(N * C),
        bytes_accessed=int(2 * N * C * HW * 4
                           + (2 * C * hid + hid * C + hid + C) * 4),
    )

    out_flat = pl.pallas_call(
        _make_body(HW),
        out_shape=jax.ShapeDtypeStruct((N, C, HW), jnp.float32),
        grid=(N // NB,),
        in_specs=[
            pl.BlockSpec((NB, C, HW), lambda n: (n, 0, 0)),
            pl.BlockSpec((C, hid), lambda n: (0, 0)),
            pl.BlockSpec((C, hid), lambda n: (0, 0)),
            pl.BlockSpec((1, hid), lambda n: (0, 0)),
            pl.BlockSpec((hid, C), lambda n: (0, 0)),
            pl.BlockSpec((1, C), lambda n: (0, 0)),
        ],
        out_specs=pl.BlockSpec((NB, C, HW), lambda n: (n, 0, 0)),
        compiler_params=pltpu.CompilerParams(
            dimension_semantics=("parallel",),
            vmem_limit_bytes=_VMEM_LIMIT_BYTES),
        cost_estimate=cost,
    )(x_flat, w1a, w1b, b1_2d, w2, b2_2d)

    return out_flat.reshape(N, C, H, W)


# speedup vs baseline: 7.1520x; 4.6007x over previous
import jax
import jax.numpy as jnp
from jax.experimental import pallas as pl
from jax.experimental.pallas import tpu as pltpu

_VMEM_LIMIT_BYTES = 64 * 1024 * 1024


def _make_body(hw):
    inv_hw = 1.0 / float(hw)

    def body(x_ref, w1a_ref, w1b_ref, b1_ref, w2_ref, b2_ref, o_ref):
        x = x_ref[...]
        avg = jnp.sum(x, axis=0) * inv_hw
        mx = jnp.max(x, axis=0)

        h = (jnp.dot(avg, w1a_ref[...], preferred_element_type=jnp.float32)
             + jnp.dot(mx, w1b_ref[...], preferred_element_type=jnp.float32)
             + b1_ref[...])
        h = jnp.maximum(h, 0.0)
        logits = (jnp.dot(h, w2_ref[...], preferred_element_type=jnp.float32)
                  + b2_ref[...])
        scale = jax.nn.sigmoid(logits)
        o_ref[...] = x * scale[None, :, :]

    return body


def kernel(x_nchw, w1, b1, w2, b2):
    N, C, H, W = x_nchw.shape
    HW = H * W
    hid = w1.shape[1]

    w1a = w1[:C, :]
    w1b = w1[C:, :]
    b1_2d = b1.reshape(1, hid)
    b2_2d = b2.reshape(1, C)

    xt = jnp.transpose(x_nchw, (2, 3, 0, 1)).reshape(HW, N, C)
    xt = xt.astype(jnp.float32)

    for nb in (8, 4, 2, 1):
        if N % nb == 0:
            NB = nb
            break

    cost = pl.CostEstimate(
        flops=int(N * (2 * 2 * C * hid + 2 * hid * C) + 2 * N * C * HW),
        transcendentals=int(N * C),
        bytes_accessed=int(2 * N * C * HW * 4
                           + (2 * C * hid + hid * C + hid + C) * 4),
    )

    out = pl.pallas_call(
        _make_body(HW),
        out_shape=jax.ShapeDtypeStruct((HW, N, C), jnp.float32),
        grid=(N // NB,),
        in_specs=[
            pl.BlockSpec((HW, NB, C), lambda n: (0, n, 0)),
            pl.BlockSpec((C, hid), lambda n: (0, 0)),
            pl.BlockSpec((C, hid), lambda n: (0, 0)),
            pl.BlockSpec((1, hid), lambda n: (0, 0)),
            pl.BlockSpec((hid, C), lambda n: (0, 0)),
            pl.BlockSpec((1, C), lambda n: (0, 0)),
        ],
        out_specs=pl.BlockSpec((HW, NB, C), lambda n: (0, n, 0)),
        compiler_params=pltpu.CompilerParams(
            dimension_semantics=("parallel",),
            vmem_limit_bytes=_VMEM_LIMIT_BYTES),
        cost_estimate=cost,
    )(xt, w1a, w1b, b1_2d, w2, b2_2d)

    return jnp.transpose(out.reshape(H, W, N, C), (2, 3, 0, 1))


# speedup vs baseline: 7.7078x; 1.0777x over previous
import jax
import jax.numpy as jnp
from jax.experimental import pallas as pl
from jax.experimental.pallas import tpu as pltpu

_VMEM_LIMIT_BYTES = 64 * 1024 * 1024


def _make_body(hw):
    inv_hw = 1.0 / float(hw)

    def body(x_ref, w1a_ref, w1b_ref, b1_ref, w2_ref, b2_ref, o_ref):
        x = x_ref[...]
        avg = jnp.sum(x, axis=0) * inv_hw
        mx = jnp.max(x, axis=0)

        h = (jnp.dot(avg, w1a_ref[...], preferred_element_type=jnp.float32)
             + jnp.dot(mx, w1b_ref[...], preferred_element_type=jnp.float32)
             + b1_ref[...])
        h = jnp.maximum(h, 0.0)
        logits = (jnp.dot(h, w2_ref[...], preferred_element_type=jnp.float32)
                  + b2_ref[...])
        scale = jax.nn.sigmoid(logits)
        o_ref[...] = x * scale[None, :, :]

    return body


def kernel(x_nchw, w1, b1, w2, b2):
    N, C, H, W = x_nchw.shape
    HW = H * W
    hid = w1.shape[1]

    w1a = w1[:C, :]
    w1b = w1[C:, :]
    b1_2d = b1.reshape(1, hid)
    b2_2d = b2.reshape(1, C)

    xt = jnp.transpose(x_nchw, (2, 3, 0, 1)).reshape(HW, N, C)
    xt = xt.astype(jnp.float32)

    for nb in (16, 8, 4, 2, 1):
        if N % nb == 0:
            NB = nb
            break

    cost = pl.CostEstimate(
        flops=int(N * (2 * 2 * C * hid + 2 * hid * C) + 2 * N * C * HW),
        transcendentals=int(N * C),
        bytes_accessed=int(2 * N * C * HW * 4
                           + (2 * C * hid + hid * C + hid + C) * 4),
    )

    out = pl.pallas_call(
        _make_body(HW),
        out_shape=jax.ShapeDtypeStruct((HW, N, C), jnp.float32),
        grid=(N // NB,),
        in_specs=[
            pl.BlockSpec((HW, NB, C), lambda n: (0, n, 0)),
            pl.BlockSpec((C, hid), lambda n: (0, 0)),
            pl.BlockSpec((C, hid), lambda n: (0, 0)),
            pl.BlockSpec((1, hid), lambda n: (0, 0)),
            pl.BlockSpec((hid, C), lambda n: (0, 0)),
            pl.BlockSpec((1, C), lambda n: (0, 0)),
        ],
        out_specs=pl.BlockSpec((HW, NB, C), lambda n: (0, n, 0)),
        compiler_params=pltpu.CompilerParams(
            dimension_semantics=("parallel",),
            vmem_limit_bytes=_VMEM_LIMIT_BYTES),
        cost_estimate=cost,
    )(xt, w1a, w1b, b1_2d, w2, b2_2d)

    return jnp.transpose(out.reshape(H, W, N, C), (2, 3, 0, 1))


# speedup vs baseline: 7.8447x; 1.0178x over previous
import jax
import jax.numpy as jnp
from jax.experimental import pallas as pl
from jax.experimental.pallas import tpu as pltpu

_VMEM_LIMIT_BYTES = 64 * 1024 * 1024


def _make_body(hw):
    inv_hw = 1.0 / float(hw)

    def body(x_ref, w1a_ref, w1b_ref, b1_ref, w2_ref, b2_ref, o_ref):
        x = x_ref[...]
        avg = jnp.sum(x, axis=0) * inv_hw
        mx = jnp.max(x, axis=0)

        h = (jnp.dot(avg, w1a_ref[...], preferred_element_type=jnp.float32)
             + jnp.dot(mx, w1b_ref[...], preferred_element_type=jnp.float32)
             + b1_ref[...])
        h = jnp.maximum(h, 0.0)
        logits = (jnp.dot(h, w2_ref[...], preferred_element_type=jnp.float32)
                  + b2_ref[...])
        scale = jax.nn.sigmoid(logits)
        o_ref[...] = x * scale[None, :, :]

    return body


def kernel(x_nchw, w1, b1, w2, b2):
    N, C, H, W = x_nchw.shape
    HW = H * W
    hid = w1.shape[1]

    w1a = w1[:C, :]
    w1b = w1[C:, :]
    b1_2d = b1.reshape(1, hid)
    b2_2d = b2.reshape(1, C)

    xt = jnp.transpose(x_nchw, (2, 3, 0, 1)).reshape(HW, N, C)
    xt = xt.astype(jnp.float32)

    for nb in (32, 16, 8, 4, 2, 1):
        if N % nb == 0:
            NB = nb
            break

    cost = pl.CostEstimate(
        flops=int(N * (2 * 2 * C * hid + 2 * hid * C) + 2 * N * C * HW),
        transcendentals=int(N * C),
        bytes_accessed=int(2 * N * C * HW * 4
                           + (2 * C * hid + hid * C + hid + C) * 4),
    )

    out = pl.pallas_call(
        _make_body(HW),
        out_shape=jax.ShapeDtypeStruct((HW, N, C), jnp.float32),
        grid=(N // NB,),
        in_specs=[
            pl.BlockSpec((HW, NB, C), lambda n: (0, n, 0)),
            pl.BlockSpec((C, hid), lambda n: (0, 0)),
            pl.BlockSpec((C, hid), lambda n: (0, 0)),
            pl.BlockSpec((1, hid), lambda n: (0, 0)),
            pl.BlockSpec((hid, C), lambda n: (0, 0)),
            pl.BlockSpec((1, C), lambda n: (0, 0)),
        ],
        out_specs=pl.BlockSpec((HW, NB, C), lambda n: (0, n, 0)),
        compiler_params=pltpu.CompilerParams(
            dimension_semantics=("parallel",),
            vmem_limit_bytes=_VMEM_LIMIT_BYTES),
        cost_estimate=cost,
    )(xt, w1a, w1b, b1_2d, w2, b2_2d)

    return jnp.transpose(out.reshape(H, W, N, C), (2, 3, 0, 1))
